# baseline (speedup 1.0000x reference)
import functools

import jax
import jax.numpy as jnp
from jax import lax
from jax.experimental import pallas as pl
from jax.experimental.pallas import tpu as pltpu
from jax.experimental.pallas import tpu_sc as plsc

VOCAB = 30522
H = 768
MAX_POS = 2048
BATCH = 4
L = 2048
EPS = 1e-12

NC = 2
NS = 16
NW = NC * NS
LPW = L // NW
CH = 32
NCH = LPW // CH
H16 = H // 16


def _rsqrt_vec(x):
    i = lax.bitcast_convert_type(x, jnp.int32)
    i = jnp.full((16,), 0x5F3759DF, jnp.int32) - lax.shift_right_arithmetic(
        i, jnp.full((16,), 1, jnp.int32))
    y = lax.bitcast_convert_type(i, jnp.float32)
    half_x = 0.5 * x
    for _ in range(3):
        y = y * (1.5 - half_x * y * y)
    return y


def _lane_sum(x):
    lanes = lax.iota(jnp.int32, 16)
    for k in (8, 4, 2, 1):
        x = x + x.at[lanes ^ k].get(mode="promise_in_bounds",
                                    unique_indices=True)
    return x


def _body(ids_hbm, seg_hbm, word_hbm, pos_hbm, type_hbm, gamma_hbm, beta_hbm,
          out_hbm,
          idx_v, segi_v, segf_v, pos_v, type_v, td_v, gamma_v, beta_v,
          wbuf0, wbuf1, sem_g0, sem_g1, sem_o0, sem_o1):
    c = lax.axis_index("c")
    s = lax.axis_index("s")
    wid = s * NC + c
    l0 = wid * LPW

    for b in range(BATCH):
        pltpu.sync_copy(ids_hbm.at[b, pl.ds(l0, LPW)], idx_v.at[b])
        pltpu.sync_copy(seg_hbm.at[b, pl.ds(l0, LPW)], segi_v.at[b])
    pltpu.sync_copy(pos_hbm.at[pl.ds(l0, LPW)], pos_v)
    pltpu.sync_copy(type_hbm, type_v)
    pltpu.sync_copy(gamma_hbm, gamma_v)
    pltpu.sync_copy(beta_hbm, beta_v)

    for b in range(BATCH):
        for j in range(LPW // 16):
            segf_v[b, pl.ds(j * 16, 16)] = segi_v[b, pl.ds(j * 16, 16)].astype(
                jnp.float32)

    def _h_setup(h, carry):
        t0 = type_v[0, pl.ds(h * 16, 16)]
        t1 = type_v[1, pl.ds(h * 16, 16)]
        td_v[pl.ds(h * 16, 16)] = t1 - t0
        return carry

    lax.fori_loop(0, H16, _h_setup, 0)

    def _l_setup(l, carry):
        def _hb(h, cc):
            pos_v[l, pl.ds(h * 16, 16)] = (
                pos_v[l, pl.ds(h * 16, 16)] + type_v[0, pl.ds(h * 16, 16)])
            return cc
        return lax.fori_loop(0, H16, _hb, carry)

    lax.fori_loop(0, LPW, _l_setup, 0)

    zeros = jnp.zeros((16,), jnp.float32)

    def _compute(b, cc, buf):
        lbase = cc * CH
        for g in range(CH // 16):
            sv = segf_v[b, pl.ds(lbase + g * 16, 16)]

            def _token(j, carry, g=g, sv=sv):
                i = g * 16 + j
                segf = sv.at[jnp.full((16,), 0, jnp.int32) + j].get(
                    mode="promise_in_bounds")

                def _h1(h, acc_pair):
                    acc, acc2 = acc_pair
                    v = (buf[i, pl.ds(h * 16, 16)]
                         + pos_v[lbase + i, pl.ds(h * 16, 16)]
                         + segf * td_v[pl.ds(h * 16, 16)])
                    buf[i, pl.ds(h * 16, 16)] = v
                    return (acc + v, acc2 + v * v)

                acc, acc2 = lax.fori_loop(0, H16, _h1, (zeros, zeros))
                mean = _lane_sum(acc) * (1.0 / H)
                var = _lane_sum(acc2) * (1.0 / H) - mean * mean
                inv = _rsqrt_vec(var + EPS)

                def _h2(h, cc2):
                    v = buf[i, pl.ds(h * 16, 16)]
                    buf[i, pl.ds(h * 16, 16)] = (
                        (v - mean) * inv * gamma_v[pl.ds(h * 16, 16)]
                        + beta_v[pl.ds(h * 16, 16)])
                    return cc2

                lax.fori_loop(0, H16, _h2, 0)
                return carry

            lax.fori_loop(0, 16, _token, 0)

    bufs = [wbuf0, wbuf1]
    gsems = [sem_g0, sem_g1]
    osems = [sem_o0, sem_o1]
    chunks = [(b, cc) for b in range(BATCH) for cc in range(NCH)]

    def _start_gather(k):
        b, cc = chunks[k]
        return pltpu.async_copy(
            word_hbm.at[idx_v.at[b, pl.ds(cc * CH, CH)]],
            bufs[k % 2], gsems[k % 2])

    def _start_out(k):
        b, cc = chunks[k]
        return pltpu.async_copy(
            bufs[k % 2], out_hbm.at[b, pl.ds(l0 + cc * CH, CH)], osems[k % 2])

    pending_out = {}
    gh = {0: _start_gather(0)}
    for k in range(len(chunks)):
        if k + 1 < len(chunks):
            if k >= 1:
                pending_out.pop(k - 1).wait()
            gh[k + 1] = _start_gather(k + 1)
        gh.pop(k).wait()
        b, cc = chunks[k]
        _compute(b, cc, bufs[k % 2])
        pending_out[k] = _start_out(k)
    pending_out.pop(len(chunks) - 2).wait()
    pending_out.pop(len(chunks) - 1).wait()


@jax.jit
def _embed(input_ids, segment_ids, word_embeddings, position_embeddings,
           token_type_embeddings, ln_gamma, ln_beta):
    mesh = plsc.VectorSubcoreMesh(core_axis_name="c", subcore_axis_name="s")
    fn = functools.partial(
        pl.kernel,
        mesh=mesh,
        out_type=jax.ShapeDtypeStruct((BATCH, L, H), jnp.float32),
        scratch_types=[
            pltpu.VMEM((BATCH, LPW), jnp.int32),
            pltpu.VMEM((BATCH, LPW), jnp.int32),
            pltpu.VMEM((BATCH, LPW), jnp.float32),
            pltpu.VMEM((LPW, H), jnp.float32),
            pltpu.VMEM((2, H), jnp.float32),
            pltpu.VMEM((H,), jnp.float32),
            pltpu.VMEM((H,), jnp.float32),
            pltpu.VMEM((H,), jnp.float32),
            pltpu.VMEM((CH, H), jnp.float32),
            pltpu.VMEM((CH, H), jnp.float32),
            pltpu.SemaphoreType.DMA,
            pltpu.SemaphoreType.DMA,
            pltpu.SemaphoreType.DMA,
            pltpu.SemaphoreType.DMA,
        ],
    )(_body)
    return fn(input_ids, segment_ids, word_embeddings, position_embeddings,
              token_type_embeddings, ln_gamma, ln_beta)


def kernel(input_ids, segment_ids, word_embeddings, position_embeddings,
           token_type_embeddings, ln_gamma, ln_beta):
    return _embed(input_ids.astype(jnp.int32), segment_ids.astype(jnp.int32),
                  word_embeddings, position_embeddings, token_type_embeddings,
                  ln_gamma, ln_beta)


# speedup vs baseline: 2.6624x; 2.6624x over previous
import functools

import jax
import jax.numpy as jnp
from jax import lax
from jax.experimental import pallas as pl
from jax.experimental.pallas import tpu as pltpu
from jax.experimental.pallas import tpu_sc as plsc

VOCAB = 30522
H = 768
BATCH = 4
L = 2048
EPS = 1e-12

NC = 2
NS = 16
NW = NC * NS
TPW = L // NW
SUB = TPW // 2

TB = 256


def _sc_gather_body(ids_hbm, word_hbm, out_hbm, idx_v, buf0, buf1,
                    sem_g0, sem_g1, sem_o0, sem_o1):
    c = lax.axis_index("c")
    s = lax.axis_index("s")
    wid = s * NC + c
    base = wid * TPW

    pltpu.sync_copy(ids_hbm.at[pl.ds(base, TPW)], idx_v)
    g0 = pltpu.async_copy(
        word_hbm.at[idx_v.at[pl.ds(0, SUB)]], buf0, sem_g0)
    g1 = pltpu.async_copy(
        word_hbm.at[idx_v.at[pl.ds(SUB, SUB)]], buf1, sem_g1)
    g0.wait()
    o0 = pltpu.async_copy(buf0, out_hbm.at[pl.ds(base, SUB)], sem_o0)
    g1.wait()
    o1 = pltpu.async_copy(buf1, out_hbm.at[pl.ds(base + SUB, SUB)], sem_o1)
    o0.wait()
    o1.wait()


def _sc_gather(ids_chunk, word_embeddings):
    fn = functools.partial(
        pl.kernel,
        mesh=plsc.VectorSubcoreMesh(core_axis_name="c", subcore_axis_name="s"),
        out_type=jax.ShapeDtypeStruct((L, H), jnp.float32),
        scratch_types=[
            pltpu.VMEM((TPW,), jnp.int32),
            pltpu.VMEM((SUB, H), jnp.float32),
            pltpu.VMEM((SUB, H), jnp.float32),
            pltpu.SemaphoreType.DMA,
            pltpu.SemaphoreType.DMA,
            pltpu.SemaphoreType.DMA,
            pltpu.SemaphoreType.DMA,
        ],
    )(_sc_gather_body)
    return fn(ids_chunk, word_embeddings)


def _tc_ln_body(w_ref, p_ref, s_ref, t_ref, g_ref, b_ref, o_ref):
    w = w_ref[...]
    p = p_ref[...]
    sg = s_ref[...]
    tt = t_ref[...]
    t0 = tt[0:1, :]
    td = tt[1:2, :] - t0
    e = w + p + t0 + sg * td
    mean = jnp.mean(e, axis=1, keepdims=True)
    cen = e - mean
    var = jnp.mean(cen * cen, axis=1, keepdims=True)
    o_ref[...] = cen * lax.rsqrt(var + EPS) * g_ref[...] + b_ref[...]


def _tc_ln(wrows, pos, segf, ttype, gamma, beta):
    grid = (L // TB,)
    return pl.pallas_call(
        _tc_ln_body,
        grid=grid,
        in_specs=[
            pl.BlockSpec((TB, H), lambda i: (i, 0)),
            pl.BlockSpec((TB, H), lambda i: (i, 0)),
            pl.BlockSpec((TB, 1), lambda i: (i, 0)),
            pl.BlockSpec((2, H), lambda i: (0, 0)),
            pl.BlockSpec((1, H), lambda i: (0, 0)),
            pl.BlockSpec((1, H), lambda i: (0, 0)),
        ],
        out_specs=pl.BlockSpec((TB, H), lambda i: (i, 0)),
        out_shape=jax.ShapeDtypeStruct((L, H), jnp.float32),
    )(wrows, pos, segf, ttype, gamma, beta)


@jax.jit
def _embed(input_ids, segment_ids, word_embeddings, position_embeddings,
           token_type_embeddings, ln_gamma, ln_beta):
    pos = position_embeddings[:L]
    gamma = ln_gamma.reshape(1, H)
    beta = ln_beta.reshape(1, H)
    outs = []
    for b in range(BATCH):
        wrows = _sc_gather(input_ids[b], word_embeddings)
        segf = segment_ids[b].astype(jnp.float32).reshape(L, 1)
        outs.append(_tc_ln(wrows, pos, segf, token_type_embeddings,
                           gamma, beta))
    return jnp.stack(outs, axis=0)


def kernel(input_ids, segment_ids, word_embeddings, position_embeddings,
           token_type_embeddings, ln_gamma, ln_beta):
    return _embed(input_ids.astype(jnp.int32), segment_ids.astype(jnp.int32),
                  word_embeddings, position_embeddings, token_type_embeddings,
                  ln_gamma, ln_beta)


# speedup vs baseline: 3.6849x; 1.3841x over previous
import functools

import jax
import jax.numpy as jnp
from jax import lax
from jax.experimental import pallas as pl
from jax.experimental.pallas import tpu as pltpu
from jax.experimental.pallas import tpu_sc as plsc

VOCAB = 30522
H = 768
BATCH = 4
L = 2048
EPS = 1e-12

NCHUNK = 4
LC = L // NCHUNK
TOK = BATCH * LC

NC = 2
NS = 16
NW = NC * NS
TPW = TOK // NW
SUB = TPW // 2
WPB = NW // BATCH

TB = LC


def _sc_gather_body(chunk, ids_hbm, word_hbm, out_hbm, idx_v, buf0, buf1,
                    sem_g0, sem_g1, sem_o0, sem_o1):
    c = lax.axis_index("c")
    s = lax.axis_index("s")
    wid = s * NC + c
    b = lax.shift_right_logical(wid, 3)
    lw = lax.bitwise_and(wid, WPB - 1)
    l_off = chunk * LC + lw * TPW
    base = wid * TPW

    pltpu.sync_copy(ids_hbm.at[b, pl.ds(l_off, TPW)], idx_v)
    g0 = pltpu.async_copy(
        word_hbm.at[idx_v.at[pl.ds(0, SUB)]], buf0, sem_g0)
    g1 = pltpu.async_copy(
        word_hbm.at[idx_v.at[pl.ds(SUB, SUB)]], buf1, sem_g1)
    g0.wait()
    o0 = pltpu.async_copy(buf0, out_hbm.at[pl.ds(base, SUB)], sem_o0)
    g1.wait()
    o1 = pltpu.async_copy(buf1, out_hbm.at[pl.ds(base + SUB, SUB)], sem_o1)
    o0.wait()
    o1.wait()


def _sc_gather(chunk, input_ids, word_embeddings):
    fn = functools.partial(
        pl.kernel,
        mesh=plsc.VectorSubcoreMesh(core_axis_name="c", subcore_axis_name="s"),
        out_type=jax.ShapeDtypeStruct((TOK, H), jnp.float32),
        scratch_types=[
            pltpu.VMEM((TPW,), jnp.int32),
            pltpu.VMEM((SUB, H), jnp.float32),
            pltpu.VMEM((SUB, H), jnp.float32),
            pltpu.SemaphoreType.DMA,
            pltpu.SemaphoreType.DMA,
            pltpu.SemaphoreType.DMA,
            pltpu.SemaphoreType.DMA,
        ],
    )(functools.partial(_sc_gather_body, chunk))
    return fn(input_ids, word_embeddings)


def _tc_ln_body(chunk, w_ref, p_ref, s_ref, prm_ref, o_ref, acc_ref=None):
    del acc_ref
    w = w_ref[...]
    p = p_ref[...]
    sg = s_ref[0].astype(jnp.float32)
    prm = prm_ref[...]
    g = prm[0:1, :]
    bb = prm[1:2, :]
    t0 = prm[2:3, :]
    td = prm[3:4, :] - t0
    e = w + p + t0 + sg * td
    mean = jnp.mean(e, axis=1, keepdims=True)
    cen = e - mean
    var = jnp.mean(cen * cen, axis=1, keepdims=True)
    o_ref[0] = cen * lax.rsqrt(var + EPS) * g + bb


def _tc_ln_body_acc(chunk, a_ref, w_ref, p_ref, s_ref, prm_ref, o_ref):
    _tc_ln_body(chunk, w_ref, p_ref, s_ref, prm_ref, o_ref, acc_ref=a_ref)


def _tc_ln(chunk, acc, wrows, pos, seg3, params):
    grid = (BATCH,)
    data_specs = [
        pl.BlockSpec((TB, H), lambda i: (i, 0)),
        pl.BlockSpec((TB, H), lambda i, c=chunk: (c, 0)),
        pl.BlockSpec((1, TB, 1), lambda i, c=chunk: (i, c, 0)),
        pl.BlockSpec((4, H), lambda i: (0, 0)),
    ]
    out_spec = pl.BlockSpec((1, TB, H), lambda i, c=chunk: (i, c, 0))
    out_shape = jax.ShapeDtypeStruct((BATCH, L, H), jnp.float32)
    if chunk == 0:
        return pl.pallas_call(
            functools.partial(_tc_ln_body, chunk),
            grid=grid, in_specs=data_specs,
            out_specs=out_spec, out_shape=out_shape,
        )(wrows, pos, seg3, params)
    return pl.pallas_call(
        functools.partial(_tc_ln_body_acc, chunk),
        grid=grid,
        in_specs=[pl.BlockSpec(memory_space=pl.ANY)] + data_specs,
        out_specs=out_spec, out_shape=out_shape,
        input_output_aliases={0: 0},
    )(acc, wrows, pos, seg3, params)


@jax.jit
def _embed(input_ids, segment_ids, word_embeddings, position_embeddings,
           token_type_embeddings, ln_gamma, ln_beta):
    pos = position_embeddings[:L]
    seg3 = segment_ids.reshape(BATCH, L, 1)
    params = jnp.concatenate(
        [ln_gamma.reshape(1, H), ln_beta.reshape(1, H),
         token_type_embeddings], axis=0)
    wrows = [_sc_gather(c, input_ids, word_embeddings) for c in range(NCHUNK)]
    out = None
    for c in range(NCHUNK):
        out = _tc_ln(c, out, wrows[c], pos, seg3, params)
    return out


def kernel(input_ids, segment_ids, word_embeddings, position_embeddings,
           token_type_embeddings, ln_gamma, ln_beta):
    return _embed(input_ids.astype(jnp.int32), segment_ids.astype(jnp.int32),
                  word_embeddings, position_embeddings, token_type_embeddings,
                  ln_gamma, ln_beta)
